# baseline (speedup 1.0000x reference)
import functools

import jax
import jax.numpy as jnp
from jax import lax
from jax.experimental import pallas as pl
from jax.experimental.pallas import tpu as pltpu
from jax.experimental.pallas import tpu_sc as plsc

F = 128
N = 10000
DEG = 32
NPW_A = 576
NPW_B = 64
NP = 16 * (NPW_A + NPW_B)
CHUNK = 4
NBUF = 4
LANES = 16
NV = F // LANES
TWO_SIGMA_SQ = 2.0 * (0.1 * 0.1)


def _perm16(vec, idx):
    return lax.gather(
        vec, idx[:, None],
        lax.GatherDimensionNumbers(offset_dims=(), collapsed_slice_dims=(0,),
                                   start_index_map=(0,)),
        slice_sizes=(1,), mode=lax.GatherScatterMode.PROMISE_IN_BOUNDS)


def _lane_allsum(vec, lane_iota):
    sixteen = jnp.full((LANES,), LANES, jnp.int32)
    for sh in (8, 4, 2, 1):
        vec = vec + _perm16(vec, lax.rem(lane_iota + sh, sixteen))
    return vec


def _sc_s_kernel(npw, base_row, xt_hbm, nbr_hbm, s_hbm, idx_v, xn_v, buf_v,
                 s_v, xn_sem, *sems):
    cid = lax.axis_index("c")
    sid = lax.axis_index("s")
    lane_iota = lax.iota(jnp.int32, LANES)
    zero = jnp.zeros((LANES,), jnp.float32)
    nchunk = npw // CHUNK
    nsuper = nchunk // NBUF

    @pl.when(cid == 0)
    def _():
        nbase = sid * npw
        pltpu.sync_copy(nbr_hbm.at[pl.ds(nbase * DEG, npw * DEG)],
                        idx_v.at[pl.ds(0, npw * DEG)])

        def gather_chunk(g, b):
            return pltpu.make_async_copy(
                xt_hbm.at[idx_v.at[pl.ds(g * (CHUNK * DEG), CHUNK * DEG)]],
                buf_v.at[b], sems[b])

        def stage_xn(ss):
            return pltpu.make_async_copy(
                xt_hbm.at[pl.ds(base_row + nbase + ss * LANES, LANES), :],
                xn_v.at[lax.rem(ss, 2)], xn_sem)

        stage_xn(0).start()
        for b in range(NBUF):
            gather_chunk(b, b).start()

        def super_body(ss, carry):
            stage_xn(ss).wait()

            @pl.when(ss + 1 < nsuper)
            def _():
                stage_xn(ss + 1).start()

            xnp = lax.rem(ss, 2)
            svec = zero
            for b in range(NBUF):
                g = ss * NBUF + b
                gather_chunk(g, b).wait()

                @pl.when(g + NBUF < nchunk)
                def _():
                    gather_chunk(g + NBUF, b).start()

                def node_body(i, sv, _b=b):
                    il = _b * CHUNK + i
                    xv = [xn_v[xnp, il, pl.ds(v * LANES, LANES)]
                          for v in range(NV)]
                    accs = [zero] * NV
                    for d in range(DEG):
                        for v in range(NV):
                            r = buf_v[_b, i * DEG + d,
                                      pl.ds(v * LANES, LANES)]
                            df = r - xv[v]
                            accs[v] = accs[v] + df * df
                    tot = ((accs[0] + accs[1]) + (accs[2] + accs[3])) + \
                          ((accs[4] + accs[5]) + (accs[6] + accs[7]))
                    stot = _lane_allsum(tot, lane_iota)
                    return sv + jnp.where(lane_iota == il, stot, 0.0)

                svec = lax.fori_loop(0, CHUNK, node_body, svec)

            s_v[pl.ds(ss * LANES, LANES)] = svec
            return carry

        lax.fori_loop(0, nsuper, super_body, 0)
        pltpu.sync_copy(s_v.at[pl.ds(0, npw)], s_hbm.at[pl.ds(nbase, npw)])


def _make_sc_call(npw, base_row):
    mesh = plsc.VectorSubcoreMesh(core_axis_name="c", subcore_axis_name="s")
    return pl.kernel(
        functools.partial(_sc_s_kernel, npw, base_row),
        out_type=jax.ShapeDtypeStruct((16 * npw,), jnp.float32),
        mesh=mesh,
        scratch_types=[
            pltpu.VMEM((npw * DEG,), jnp.int32),
            pltpu.VMEM((2, LANES, F), jnp.float32),
            pltpu.VMEM((NBUF, CHUNK * DEG, F), jnp.float32),
            pltpu.VMEM((npw,), jnp.float32),
            pltpu.SemaphoreType.DMA,
        ] + [pltpu.SemaphoreType.DMA] * NBUF,
    )


@jax.jit
def _sc_s(xt, nbr_flat):
    na = 16 * NPW_A
    s_a = _make_sc_call(NPW_A, 0)(xt, nbr_flat[:na * DEG])
    s_b = _make_sc_call(NPW_B, na)(xt, nbr_flat[na * DEG:])
    return s_a, s_b


def _combine_kernel(w_ref, x_ref, s_ref, o_ref):
    w0 = w_ref[0, 0]
    w1 = w_ref[0, 1]
    kern = jnp.exp(s_ref[...] * (-1.0 / TWO_SIGMA_SQ))
    o_ref[...] = jnp.maximum(x_ref[...], 0.0) * w0 + kern * w1


@jax.jit
def _combine(x2, s, weight):
    bn = 1280
    grid = NP // bn
    return pl.pallas_call(
        _combine_kernel,
        grid=(grid,),
        in_specs=[
            pl.BlockSpec(memory_space=pltpu.SMEM),
            pl.BlockSpec((F, bn), lambda i: (0, i)),
            pl.BlockSpec((1, bn), lambda i: (0, i)),
        ],
        out_specs=pl.BlockSpec((F, bn), lambda i: (0, i)),
        out_shape=jax.ShapeDtypeStruct((F, NP), jnp.float32),
    )(weight, x2, s)


def kernel(x, neighborhood, weight):
    b, f, n = x.shape
    assert (b, f, n) == (1, F, N) and neighborhood.shape == (N, DEG)
    x2 = x.reshape(F, N)
    xt = jnp.pad(x2.T, ((0, NP - N), (0, 0)))
    nbr_flat = jnp.pad(neighborhood.astype(jnp.int32),
                       ((0, NP - N), (0, 0))).reshape(NP * DEG)
    s_a, s_b = _sc_s(xt, nbr_flat)
    s = jnp.concatenate([s_a, s_b])
    x2p = jnp.pad(x2, ((0, 0), (0, NP - N)))
    out = _combine(x2p, s.reshape(1, NP), weight)
    return out[:, :N].reshape(1, F, N)


# speedup vs baseline: 4.2022x; 4.2022x over previous
import jax
import jax.numpy as jnp
from jax import lax
from jax.experimental import pallas as pl
from jax.experimental.pallas import tpu as pltpu
from jax.experimental.pallas import tpu_sc as plsc

F = 128
N = 10000
DEG = 32
NPW = 320
NP = 32 * NPW
CHUNK = 4
NBUF = 2
GROUP = NBUF * CHUNK
LANES = 16
NV = F // LANES
TWO_SIGMA_SQ = 2.0 * (0.1 * 0.1)


def _perm16(vec, idx):
    return lax.gather(
        vec, idx[:, None],
        lax.GatherDimensionNumbers(offset_dims=(), collapsed_slice_dims=(0,),
                                   start_index_map=(0,)),
        slice_sizes=(1,), mode=lax.GatherScatterMode.PROMISE_IN_BOUNDS)


def _lane_allsum(vec, lane_iota):
    sixteen = jnp.full((LANES,), LANES, jnp.int32)
    for sh in (8, 4, 2, 1):
        vec = vec + _perm16(vec, lax.rem(lane_iota + sh, sixteen))
    return vec


def _sc_s_kernel(xt_hbm, nbr_hbm, s_hbm, idx_v, xn_v, buf_v, s_v, tab_sh,
                 xn_sem, *sems):
    cid = lax.axis_index("c")
    sid = lax.axis_index("s")
    lane_iota = lax.iota(jnp.int32, LANES)
    zero = jnp.zeros((LANES,), jnp.float32)

    rpt = NP // 16
    pltpu.sync_copy(xt_hbm.at[pl.ds(sid * rpt, rpt), :],
                    tab_sh.at[pl.ds(sid * rpt, rpt), :])
    plsc.subcore_barrier()

    def run_side(nbase, npw):
        nchunk = npw // CHUNK
        nsuper = nchunk // NBUF
        pltpu.sync_copy(nbr_hbm.at[pl.ds(nbase * DEG, npw * DEG)],
                        idx_v.at[pl.ds(0, npw * DEG)])
        for grp in range(npw // LANES):
            s_v[pl.ds(grp * LANES, LANES)] = zero

        def gather_chunk(g, b):
            return pltpu.make_async_copy(
                tab_sh.at[idx_v.at[pl.ds(g * (CHUNK * DEG), CHUNK * DEG)]],
                buf_v.at[b], sems[b])

        def stage_xn(ss):
            return pltpu.make_async_copy(
                xt_hbm.at[pl.ds(nbase + ss * GROUP, GROUP), :],
                xn_v.at[lax.rem(ss, 2)], xn_sem)

        stage_xn(0).start()
        for b in range(NBUF):
            gather_chunk(b, b).start()

        def super_body(ss, carry):
            stage_xn(ss).wait()

            @pl.when(ss + 1 < nsuper)
            def _():
                stage_xn(ss + 1).start()

            xnp = lax.rem(ss, 2)
            svec = zero
            for b in range(NBUF):
                g = ss * NBUF + b
                gather_chunk(g, b).wait()

                @pl.when(g + NBUF < nchunk)
                def _():
                    gather_chunk(g + NBUF, b).start()

                def node_body(i, sv, _b=b):
                    il = _b * CHUNK + i
                    xv = [xn_v[xnp, il, pl.ds(v * LANES, LANES)]
                          for v in range(NV)]
                    accs = [zero] * NV
                    for d in range(DEG):
                        for v in range(NV):
                            r = buf_v[_b, i * DEG + d,
                                      pl.ds(v * LANES, LANES)]
                            df = r - xv[v]
                            accs[v] = accs[v] + df * df
                    tot = ((accs[0] + accs[1]) + (accs[2] + accs[3])) + \
                          ((accs[4] + accs[5]) + (accs[6] + accs[7]))
                    stot = _lane_allsum(tot, lane_iota)
                    lane = lax.rem(ss, 2) * GROUP + il
                    return sv + jnp.where(lane_iota == lane, stot, 0.0)

                svec = lax.fori_loop(0, CHUNK, node_body, svec)

            gbase = (ss // 2) * LANES
            s_v[pl.ds(gbase, LANES)] = s_v[pl.ds(gbase, LANES)] + svec
            return carry

        lax.fori_loop(0, nsuper, super_body, 0)
        pltpu.sync_copy(s_v.at[pl.ds(0, npw)], s_hbm.at[pl.ds(nbase, npw)])

    wid = sid * 2 + cid
    run_side(wid * NPW, NPW)


@jax.jit
def _sc_s(xt, nbr_flat):
    mesh = plsc.VectorSubcoreMesh(core_axis_name="c", subcore_axis_name="s")
    return pl.kernel(
        _sc_s_kernel,
        out_type=jax.ShapeDtypeStruct((NP,), jnp.float32),
        mesh=mesh,
        scratch_types=[
            pltpu.VMEM((NPW * DEG,), jnp.int32),
            pltpu.VMEM((2, GROUP, F), jnp.float32),
            pltpu.VMEM((NBUF, CHUNK * DEG, F), jnp.float32),
            pltpu.VMEM((NPW,), jnp.float32),
            pltpu.VMEM_SHARED((NP, F), jnp.float32),
            pltpu.SemaphoreType.DMA,
        ] + [pltpu.SemaphoreType.DMA] * NBUF,
    )(xt, nbr_flat)


def _combine_kernel(w_ref, x_ref, s_ref, o_ref):
    w0 = w_ref[0, 0]
    w1 = w_ref[0, 1]
    kern = jnp.exp(s_ref[...] * (-1.0 / TWO_SIGMA_SQ))
    o_ref[...] = jnp.maximum(x_ref[...], 0.0) * w0 + kern * w1


@jax.jit
def _combine(x2, s, weight):
    bn = 1280
    grid = NP // bn
    return pl.pallas_call(
        _combine_kernel,
        grid=(grid,),
        in_specs=[
            pl.BlockSpec(memory_space=pltpu.SMEM),
            pl.BlockSpec((F, bn), lambda i: (0, i)),
            pl.BlockSpec((1, bn), lambda i: (0, i)),
        ],
        out_specs=pl.BlockSpec((F, bn), lambda i: (0, i)),
        out_shape=jax.ShapeDtypeStruct((F, NP), jnp.float32),
    )(weight, x2, s)


def kernel(x, neighborhood, weight):
    b, f, n = x.shape
    assert (b, f, n) == (1, F, N) and neighborhood.shape == (N, DEG)
    x2 = x.reshape(F, N)
    xt = jnp.pad(x2.T, ((0, NP - N), (0, 0)))
    nbr_flat = jnp.pad(neighborhood.astype(jnp.int32),
                       ((0, NP - N), (0, 0))).reshape(NP * DEG)
    s = _sc_s(xt, nbr_flat)
    x2p = jnp.pad(x2, ((0, 0), (0, NP - N)))
    out = _combine(x2p, s.reshape(1, NP), weight)
    return out[:, :N].reshape(1, F, N)
